# baseline (speedup 1.0000x reference)
import jax
import jax.numpy as jnp
from jax.experimental import pallas as pl
from jax.experimental.pallas import tpu as pltpu


def _copy_body(img_ref, out_ref):
    out_ref[...] = img_ref[...]


def kernel(img_tensor):
    B, C, H, W = img_tensor.shape
    flat = img_tensor.reshape(B * C, H, W)
    out = pl.pallas_call(
        _copy_body,
        grid=(B * C, 4),
        in_specs=[pl.BlockSpec((1, H // 4, W), lambda i, j: (i, j, 0))],
        out_specs=pl.BlockSpec((1, H // 4, W), lambda i, j: (i, j, 0)),
        out_shape=jax.ShapeDtypeStruct((B * C, H, W), jnp.float32),
        compiler_params=pltpu.CompilerParams(
            dimension_semantics=("parallel", "parallel"),
        ),
    )(flat)
    return out.reshape(B, C, H, W)


# speedup vs baseline: 1.9296x; 1.9296x over previous
import jax
import jax.numpy as jnp
from jax.experimental import pallas as pl
from jax.experimental.pallas import tpu as pltpu

_BLK_H = 512
_FIX_H = 8
_THRESHOLD = 0.5


def _fused_body(img_ref, out_ref, acc_ref):
    c = pl.program_id(1)
    img = img_ref[0]
    out_ref[0] = img
    r = img[0]
    g = img[1]
    b = img[2]
    gray = 0.2989 * r + 0.587 * g + 0.114 * b
    black = gray < _THRESHOLD
    h, w = gray.shape
    rows = jax.lax.broadcasted_iota(jnp.int32, (h, w), 0) + c * _BLK_H
    cols = jax.lax.broadcasted_iota(jnp.int32, (h, w), 1)
    y1 = jnp.max(jnp.where(black, rows, -1))
    nx0 = jnp.max(jnp.where(black, -cols, -w))
    x1 = jnp.max(jnp.where(black, cols, -1))
    lane = jax.lax.broadcasted_iota(jnp.int32, (1, 128), 1)
    vec = jnp.where(lane == 0, y1, jnp.where(lane == 1, nx0, x1))

    @pl.when(c == 0)
    def _():
        acc_ref[0] = vec

    @pl.when(c != 0)
    def _():
        acc_ref[0] = jnp.maximum(acc_ref[0], vec)


def _fix_block_idx(b, j, s_ref, h_blocks):
    y1 = s_ref[b * 3]
    return jnp.clip((y1 - 2) // _FIX_H + j, 0, h_blocks - 1)


def _fixup_body(s_ref, buf_ref, out_ref):
    b = pl.program_id(0)
    j = pl.program_id(1)
    y1 = s_ref[b * 3]
    x0 = -s_ref[b * 3 + 1]
    x1 = s_ref[b * 3 + 2]
    y_lo = jnp.maximum(y1 - 3, 0)
    _, _, h, w = buf_ref.shape
    n_blocks = 512 // _FIX_H
    rblk = jnp.clip((y1 - 2) // _FIX_H + j, 0, n_blocks - 1)
    rows = jax.lax.broadcasted_iota(jnp.int32, (h, w), 0) + rblk * _FIX_H
    cols = jax.lax.broadcasted_iota(jnp.int32, (h, w), 1)
    m = (rows <= y1) & (rows > y_lo) & (cols >= x0) & (cols < x1)
    out_ref[0] = jnp.where(m[None], 0.0, buf_ref[0])


def kernel(img_tensor):
    B, C, H, W = img_tensor.shape
    n_chunks = H // _BLK_H

    copied, acc = pl.pallas_call(
        _fused_body,
        grid=(B, n_chunks),
        in_specs=[
            pl.BlockSpec((1, C, _BLK_H, W), lambda b, c: (b, 0, c, 0)),
        ],
        out_specs=[
            pl.BlockSpec((1, C, _BLK_H, W), lambda b, c: (b, 0, c, 0)),
            pl.BlockSpec((1, 1, 128), lambda b, c: (b, 0, 0)),
        ],
        out_shape=[
            jax.ShapeDtypeStruct((B, C, H, W), jnp.float32),
            jax.ShapeDtypeStruct((B, 1, 128), jnp.int32),
        ],
        compiler_params=pltpu.CompilerParams(
            dimension_semantics=("parallel", "arbitrary"),
        ),
    )(img_tensor)

    coords = acc[:, 0, :3].reshape(-1)
    h_blocks = H // _FIX_H

    out = pl.pallas_call(
        _fixup_body,
        grid_spec=pltpu.PrefetchScalarGridSpec(
            num_scalar_prefetch=1,
            grid=(B, 2),
            in_specs=[
                pl.BlockSpec(
                    (1, C, _FIX_H, W),
                    lambda b, j, s: (b, 0, _fix_block_idx(b, j, s, h_blocks), 0),
                ),
            ],
            out_specs=pl.BlockSpec(
                (1, C, _FIX_H, W),
                lambda b, j, s: (b, 0, _fix_block_idx(b, j, s, h_blocks), 0),
            ),
        ),
        out_shape=jax.ShapeDtypeStruct((B, C, H, W), jnp.float32),
        input_output_aliases={1: 0},
        compiler_params=pltpu.CompilerParams(
            dimension_semantics=("arbitrary", "arbitrary"),
        ),
    )(coords, copied)
    return out


# speedup vs baseline: 2.4907x; 1.2907x over previous
import jax
import jax.numpy as jnp
from jax.experimental import pallas as pl
from jax.experimental.pallas import tpu as pltpu

_BLK_H = 512
_FIXROWS = 16
_THRESHOLD = 0.5


def _fused_body(img_ref, out_ref, acc_ref):
    c = pl.program_id(1)
    img = img_ref[0]
    out_ref[0] = img
    r = img[0]
    g = img[1]
    b = img[2]
    gray = 0.2989 * r + 0.587 * g + 0.114 * b
    black = gray < _THRESHOLD
    h, w = gray.shape
    rows = jax.lax.broadcasted_iota(jnp.int32, (h, w), 0) + c * _BLK_H
    cols = jax.lax.broadcasted_iota(jnp.int32, (h, w), 1)
    y1 = jnp.max(jnp.where(black, rows, -1))
    nx0 = jnp.max(jnp.where(black, -cols, -w))
    x1 = jnp.max(jnp.where(black, cols, -1))
    lane = jax.lax.broadcasted_iota(jnp.int32, (1, 128), 1)
    vec = jnp.where(lane == 0, y1, jnp.where(lane == 1, nx0, x1))

    @pl.when(c == 0)
    def _():
        acc_ref[0] = vec

    @pl.when(c != 0)
    def _():
        acc_ref[0] = jnp.maximum(acc_ref[0], vec)


def _fixup_body(s_ref, img_hbm, alias_hbm, out_hbm, buf, in_sems, out_sems):
    del alias_hbm
    B, C, H, W = img_hbm.shape

    def window(b):
        y1 = s_ref[3 * b]
        return pl.multiple_of(jnp.clip(((y1 - 2) // 8) * 8, 0, H - _FIXROWS), 8)

    for b in range(B):
        ys = window(b)
        pltpu.make_async_copy(
            img_hbm.at[b, :, pl.ds(ys, _FIXROWS), :], buf.at[b], in_sems.at[b]
        ).start()
    for b in range(B):
        y1 = s_ref[3 * b]
        x0 = -s_ref[3 * b + 1]
        x1 = s_ref[3 * b + 2]
        y_lo = jnp.maximum(y1 - 3, 0)
        ys = window(b)
        pltpu.make_async_copy(
            img_hbm.at[b, :, pl.ds(ys, _FIXROWS), :], buf.at[b], in_sems.at[b]
        ).wait()
        rows = jax.lax.broadcasted_iota(jnp.int32, (_FIXROWS, W), 0) + ys
        cols = jax.lax.broadcasted_iota(jnp.int32, (_FIXROWS, W), 1)
        m = (rows <= y1) & (rows > y_lo) & (cols >= x0) & (cols < x1)
        buf[b] = jnp.where(m[None], 0.0, buf[b])
        pltpu.make_async_copy(
            buf.at[b], out_hbm.at[b, :, pl.ds(ys, _FIXROWS), :], out_sems.at[b]
        ).start()
    for b in range(B):
        ys = window(b)
        pltpu.make_async_copy(
            buf.at[b], out_hbm.at[b, :, pl.ds(ys, _FIXROWS), :], out_sems.at[b]
        ).wait()


def kernel(img_tensor):
    B, C, H, W = img_tensor.shape
    n_chunks = H // _BLK_H

    copied, acc = pl.pallas_call(
        _fused_body,
        grid=(B, n_chunks),
        in_specs=[
            pl.BlockSpec((1, C, _BLK_H, W), lambda b, c: (b, 0, c, 0)),
        ],
        out_specs=[
            pl.BlockSpec((1, C, _BLK_H, W), lambda b, c: (b, 0, c, 0)),
            pl.BlockSpec((1, 1, 128), lambda b, c: (b, 0, 0)),
        ],
        out_shape=[
            jax.ShapeDtypeStruct((B, C, H, W), jnp.float32),
            jax.ShapeDtypeStruct((B, 1, 128), jnp.int32),
        ],
        compiler_params=pltpu.CompilerParams(
            dimension_semantics=("parallel", "arbitrary"),
        ),
    )(img_tensor)

    coords = acc[:, 0, :3].reshape(-1)

    out = pl.pallas_call(
        _fixup_body,
        grid_spec=pltpu.PrefetchScalarGridSpec(
            num_scalar_prefetch=1,
            grid=(1,),
            in_specs=[
                pl.BlockSpec(memory_space=pl.ANY),
                pl.BlockSpec(memory_space=pl.ANY),
            ],
            out_specs=pl.BlockSpec(memory_space=pl.ANY),
            scratch_shapes=[
                pltpu.VMEM((B, C, _FIXROWS, W), jnp.float32),
                pltpu.SemaphoreType.DMA((B,)),
                pltpu.SemaphoreType.DMA((B,)),
            ],
        ),
        out_shape=jax.ShapeDtypeStruct((B, C, H, W), jnp.float32),
        input_output_aliases={2: 0},
    )(coords, img_tensor, copied)
    return out


# speedup vs baseline: 3.1671x; 1.2716x over previous
import jax
import jax.numpy as jnp
from jax.experimental import pallas as pl
from jax.experimental.pallas import tpu as pltpu

_CH = 128
_NBUF = 8
_FIXROWS = 16
_THRESHOLD = 0.5


def _body(img_hbm, out_hbm, bufs, fixbufs, in_sems, out_sems, fin_sems, fout_sems):
    B, C, H, W = img_hbm.shape
    n_per_img = H // _CH
    n_chunks = B * n_per_img

    def chunk_in(k):
        b, j = divmod(k, n_per_img)
        slot = k % _NBUF
        return pltpu.make_async_copy(
            img_hbm.at[b, :, pl.ds(j * _CH, _CH), :], bufs.at[slot], in_sems.at[slot]
        )

    def chunk_out(k):
        b, j = divmod(k, n_per_img)
        slot = k % _NBUF
        return pltpu.make_async_copy(
            bufs.at[slot], out_hbm.at[b, :, pl.ds(j * _CH, _CH), :], out_sems.at[slot]
        )

    def fix_window(scal):
        return pl.multiple_of(
            jnp.clip(((scal[0] - 2) // 8) * 8, 0, H - _FIXROWS), 8
        )

    def fix_in(b, scal):
        ys = fix_window(scal)
        return pltpu.make_async_copy(
            img_hbm.at[b, :, pl.ds(ys, _FIXROWS), :],
            fixbufs.at[b % 2],
            fin_sems.at[b % 2],
        )

    def fix_out(b, scal):
        ys = fix_window(scal)
        return pltpu.make_async_copy(
            fixbufs.at[b % 2],
            out_hbm.at[b, :, pl.ds(ys, _FIXROWS), :],
            fout_sems.at[b % 2],
        )

    def mask_fixbuf(b, scal):
        y1, nx0, x1 = scal
        x0 = -nx0
        y_lo = jnp.maximum(y1 - 3, 0)
        ys = fix_window(scal)
        rows = jax.lax.broadcasted_iota(jnp.int32, (_FIXROWS, W), 0) + ys
        cols = jax.lax.broadcasted_iota(jnp.int32, (_FIXROWS, W), 1)
        m = (rows <= y1) & (rows > y_lo) & (cols >= x0) & (cols < x1)
        fixbufs[b % 2] = jnp.where(m[None], 0.0, fixbufs[b % 2])

    def reduce_chunk(k):
        b, j = divmod(k, n_per_img)
        slot = k % _NBUF
        gray = (
            0.2989 * bufs[slot, 0]
            + 0.587 * bufs[slot, 1]
            + 0.114 * bufs[slot, 2]
        )
        black = gray < _THRESHOLD
        rows = jax.lax.broadcasted_iota(jnp.int32, (_CH, W), 0) + j * _CH
        cols = jax.lax.broadcasted_iota(jnp.int32, (_CH, W), 1)
        y1 = jnp.max(jnp.where(black, rows, -1))
        nx0 = jnp.max(jnp.where(black, -cols, -W))
        x1 = jnp.max(jnp.where(black, cols, -1))
        return y1, nx0, x1

    out_waited = [False] * n_chunks

    def wait_out(k):
        if not out_waited[k]:
            chunk_out(k).wait()
            out_waited[k] = True

    scals = {}
    partial = None

    for k in range(min(_NBUF, n_chunks)):
        chunk_in(k).start()

    for k in range(n_chunks):
        b, j = divmod(k, n_per_img)
        kp = k + _NBUF // 2
        if _NBUF <= kp < n_chunks:
            wait_out(kp - _NBUF)
            chunk_in(kp).start()

        chunk_in(k).wait()
        t = reduce_chunk(k)
        partial = t if j == 0 else tuple(map(jnp.maximum, partial, t))
        chunk_out(k).start()

        if j == n_per_img - 1:
            scals[b] = partial
            if b >= 2:
                fix_out(b - 2, scals[b - 2]).wait()
            fix_in(b, scals[b]).start()
            if b >= 1:
                fix_in(b - 1, scals[b - 1]).wait()
                mask_fixbuf(b - 1, scals[b - 1])
                for kk in range((b - 1) * n_per_img, b * n_per_img):
                    wait_out(kk)
                fix_out(b - 1, scals[b - 1]).start()

    for k in range(n_chunks):
        wait_out(k)
    fix_in(B - 1, scals[B - 1]).wait()
    mask_fixbuf(B - 1, scals[B - 1])
    fix_out(B - 1, scals[B - 1]).start()
    fix_out(B - 2, scals[B - 2]).wait()
    fix_out(B - 1, scals[B - 1]).wait()


def kernel(img_tensor):
    B, C, H, W = img_tensor.shape
    return pl.pallas_call(
        _body,
        in_specs=[pl.BlockSpec(memory_space=pl.ANY)],
        out_specs=pl.BlockSpec(memory_space=pl.ANY),
        out_shape=jax.ShapeDtypeStruct((B, C, H, W), jnp.float32),
        scratch_shapes=[
            pltpu.VMEM((_NBUF, C, _CH, W), jnp.float32),
            pltpu.VMEM((2, C, _FIXROWS, W), jnp.float32),
            pltpu.SemaphoreType.DMA((_NBUF,)),
            pltpu.SemaphoreType.DMA((_NBUF,)),
            pltpu.SemaphoreType.DMA((2,)),
            pltpu.SemaphoreType.DMA((2,)),
        ],
    )(img_tensor)
